# Initial kernel scaffold; baseline (speedup 1.0000x reference)
import jax
import jax.numpy as jnp
from jax.experimental import pallas as pl


def kernel(pos, pos0, batch):
    raise NotImplementedError("write your pallas kernel here")


# speedup vs baseline: 1.1824x; 1.1824x over previous
import functools

import numpy as np
import jax
import jax.numpy as jnp
from jax import lax
from jax.experimental import pallas as pl
from jax.experimental.pallas import tpu as pltpu
from jax.experimental.pallas import tpu_sc as plsc

_N = 800000
_G = 4096
_A = 1.0
_D = 1.0
_KY = 1.0
_KZ = 1.0
_D2 = _D * _D

_LANES = 16
_BLK_GROUPS = 250
_BLK_ATOMS = _BLK_GROUPS * _LANES
_BLK_FLOATS = _BLK_ATOMS * 3
_NUM_BLOCKS = _N // _BLK_ATOMS
_NW = 32
_MAX_BLKS_PER_TILE = -(-_NUM_BLOCKS // _NW)


def _phase_consts():
    cx, ch, gx, gc, ao = [], [], [], [], []
    for ph in range(3):
        comp = [(16 * ph + l) % 3 for l in range(16)]
        cx.append([_A if c == 0 else 0.0 for c in comp])
        ch.append([0.0 if c == 0 else (0.5 * _KY if c == 1 else 0.5 * _KZ)
                   for c in comp])
        gx.append([-4.0 * _A if c == 0 else 0.0 for c in comp])
        gc.append([0.0 if c == 0 else (-_KY if c == 1 else -_KZ)
                   for c in comp])
        ao.append([(16 * ph + l) // 3 for l in range(16)])
    asf = lambda a: [np.asarray(v, np.float32) for v in a]
    return (asf(cx), asf(ch), asf(gx), asf(gc),
            [np.asarray(v, np.int32) for v in ao])


_CX, _CH, _GX, _GC, _AO = _phase_consts()


def _sc_body(pos_hbm, pos0_hbm, batch_hbm, f_hbm, part_hbm,
             posb, pos0b, fb, batchb, acc):
    c = lax.axis_index("c")
    s = lax.axis_index("s")
    wid = c * 16 + s

    zero = jnp.zeros((_LANES,), jnp.float32)

    def zbody(i, carry):
        acc[pl.ds(i * _LANES, _LANES)] = zero
        return carry

    lax.fori_loop(0, _G // _LANES, zbody, 0)

    lane = lax.iota(jnp.int32, _LANES)
    cx, ch, gx, gc, ao = [], [], [], [], []
    for ph in range(3):
        flat = lane + (16 * ph)
        comp = lax.rem(flat, 3)
        isx = comp == 0
        isy = comp == 1
        cx.append(jnp.where(isx, jnp.float32(_A), jnp.float32(0.0)))
        ch.append(jnp.where(isx, jnp.float32(0.0),
                            jnp.where(isy, jnp.float32(0.5 * _KY),
                                      jnp.float32(0.5 * _KZ))))
        gx.append(jnp.where(isx, jnp.float32(-4.0 * _A), jnp.float32(0.0)))
        gc.append(jnp.where(isx, jnp.float32(0.0),
                            jnp.where(isy, jnp.float32(-_KY),
                                      jnp.float32(-_KZ))))
        ao.append(lax.div(flat, jnp.int32(3)))

    for k in range(_MAX_BLKS_PER_TILE):
        blk = wid + k * _NW

        @pl.when(blk < _NUM_BLOCKS)
        def _process():
            fbase = blk * _BLK_FLOATS
            abase = blk * _BLK_ATOMS
            pltpu.sync_copy(pos_hbm.at[pl.ds(fbase, _BLK_FLOATS)], posb)
            pltpu.sync_copy(pos0_hbm.at[pl.ds(fbase, _BLK_FLOATS)], pos0b)
            pltpu.sync_copy(batch_hbm.at[pl.ds(abase, _BLK_ATOMS)], batchb)

            def gbody(j, carry):
                base = j * 48
                for ph in range(3):
                    off = base + ph * _LANES
                    p = posb[pl.ds(off, _LANES)]
                    q = pos0b[pl.ds(off, _LANES)]
                    dr = p - q
                    t = dr * dr
                    u = t - _D2
                    e = cx[ph] * (u * u) + ch[ph] * t
                    f = dr * (gx[ph] * u + gc[ph])
                    fb[pl.ds(off, _LANES)] = f
                    idx = ao[ph] + j * _LANES
                    g = plsc.load_gather(batchb, [idx])
                    plsc.addupdate_scatter(acc, [g], e)
                return carry

            lax.fori_loop(0, _BLK_GROUPS, gbody, 0)
            pltpu.sync_copy(fb, f_hbm.at[pl.ds(fbase, _BLK_FLOATS)])

    pltpu.sync_copy(acc, part_hbm.at[wid])


def _reduce_body(p_ref, o_ref):
    o_ref[...] = jnp.sum(p_ref[...], axis=0, keepdims=True)


def kernel(pos, pos0, batch):
    posf = pos.reshape(-1)
    pos0f = pos0.reshape(-1)
    mesh = plsc.VectorSubcoreMesh(core_axis_name="c", subcore_axis_name="s")
    forces_flat, parts = pl.kernel(
        _sc_body,
        mesh=mesh,
        compiler_params=pltpu.CompilerParams(needs_layout_passes=False),
        out_type=[
            jax.ShapeDtypeStruct((_N * 3,), jnp.float32),
            jax.ShapeDtypeStruct((_NW, _G), jnp.float32),
        ],
        scratch_types=[
            pltpu.VMEM((_BLK_FLOATS,), jnp.float32),
            pltpu.VMEM((_BLK_FLOATS,), jnp.float32),
            pltpu.VMEM((_BLK_FLOATS,), jnp.float32),
            pltpu.VMEM((_BLK_ATOMS,), jnp.int32),
            pltpu.VMEM((_G,), jnp.float32),
        ],
    )(posf, pos0f, batch)
    energy2d = pl.pallas_call(
        _reduce_body,
        out_shape=jax.ShapeDtypeStruct((1, _G), jnp.float32),
    )(parts)
    return energy2d.reshape(_G), forces_flat.reshape(_N, 3)


# speedup vs baseline: 7.0225x; 5.9390x over previous
import numpy as np
import jax
import jax.numpy as jnp
from jax import lax
from jax.experimental import pallas as pl
from jax.experimental.pallas import tpu as pltpu
from jax.experimental.pallas import tpu_sc as plsc

_N = 800000
_G = 4096
_A = 1.0
_D = 1.0
_KY = 1.0
_KZ = 1.0
_D2 = _D * _D

_LANES = 16
_BR = 16000
_TC_GRID = _N // _BR
_EROWS = _BR // 128
_NW = 32
_MAX_BLKS_PER_TILE = -(-_TC_GRID // _NW)


def _tc_ew_body(p_ref, q_ref, f_ref, e_ref):
    dr = p_ref[...] - q_ref[...]
    t = dr * dr
    u = t - _D2
    col = lax.broadcasted_iota(jnp.int32, (1, 3), 1)
    isx = col == 0
    isy = col == 1
    fz = jnp.float32(0.0)
    cxr = jnp.where(isx, jnp.float32(_A), fz)
    chr_ = jnp.where(isx, fz, jnp.where(isy, jnp.float32(0.5 * _KY),
                                        jnp.float32(0.5 * _KZ)))
    gxr = jnp.where(isx, jnp.float32(-4.0 * _A), fz)
    gcr = jnp.where(isx, fz, jnp.where(isy, jnp.float32(-_KY),
                                       jnp.float32(-_KZ)))
    f_ref[...] = dr * (gxr * u + gcr)
    e = jnp.sum(cxr * (u * u) + chr_ * t, axis=1)
    e_ref[...] = e.reshape(1, _EROWS, 128)


def _sc_seg_body(e_hbm, b_hbm, part_hbm, eb, bb, acc):
    c = lax.axis_index("c")
    s = lax.axis_index("s")
    wid = c * 16 + s

    zero = jnp.zeros((_LANES,), jnp.float32)

    def zbody(i, carry):
        acc[pl.ds(i * _LANES, _LANES)] = zero
        return carry

    lax.fori_loop(0, _G // _LANES, zbody, 0)

    for k in range(_MAX_BLKS_PER_TILE):
        blk = wid + k * _NW

        @pl.when(blk < _TC_GRID)
        def _process():
            pltpu.sync_copy(e_hbm.at[blk], eb)
            pltpu.sync_copy(b_hbm.at[pl.ds(blk * _BR, _BR)], bb)

            def rbody(r, carry):
                for c8 in range(8):
                    ev = eb[r, pl.ds(c8 * _LANES, _LANES)]
                    gv = bb[pl.ds(r * 128 + c8 * _LANES, _LANES)]
                    plsc.addupdate_scatter(acc, [gv], ev)
                return carry

            lax.fori_loop(0, _EROWS, rbody, 0)

    pltpu.sync_copy(acc, part_hbm.at[wid])


def _reduce_body(p_ref, o_ref):
    o_ref[...] = jnp.sum(p_ref[...], axis=0, keepdims=True)


def kernel(pos, pos0, batch):
    forces, e3d = pl.pallas_call(
        _tc_ew_body,
        grid=(_TC_GRID,),
        in_specs=[
            pl.BlockSpec((_BR, 3), lambda i: (i, 0)),
            pl.BlockSpec((_BR, 3), lambda i: (i, 0)),
        ],
        out_specs=[
            pl.BlockSpec((_BR, 3), lambda i: (i, 0)),
            pl.BlockSpec((1, _EROWS, 128), lambda i: (i, 0, 0)),
        ],
        out_shape=[
            jax.ShapeDtypeStruct((_N, 3), jnp.float32),
            jax.ShapeDtypeStruct((_TC_GRID, _EROWS, 128), jnp.float32),
        ],
    )(pos, pos0)

    mesh = plsc.VectorSubcoreMesh(core_axis_name="c", subcore_axis_name="s")
    parts = pl.kernel(
        _sc_seg_body,
        mesh=mesh,
        compiler_params=pltpu.CompilerParams(needs_layout_passes=False),
        out_type=jax.ShapeDtypeStruct((_NW, _G), jnp.float32),
        scratch_types=[
            pltpu.VMEM((_EROWS, 128), jnp.float32),
            pltpu.VMEM((_BR,), jnp.int32),
            pltpu.VMEM((_G,), jnp.float32),
        ],
    )(e3d, batch)

    energy2d = pl.pallas_call(
        _reduce_body,
        out_shape=jax.ShapeDtypeStruct((1, _G), jnp.float32),
    )(parts)
    return energy2d.reshape(_G), forces
